# baseline (speedup 1.0000x reference)
import functools

import jax
import jax.numpy as jnp
from jax.experimental import pallas as pl
from jax.experimental.pallas import tpu as pltpu

N = 50000
E = 800000
M = 12500
D = 64

E_BLK = 2000
M_BLK = 2048


def _lrelu(v):
    return jnp.where(v > 0, v, 0.01 * v)


def _edge_mlp_body(xg_ref, dp_ref, w1x_ref, w1p_ref, b1_ref, w2_ref, b2_ref,
                   w3_ref, b3_ref, h_ref):
    z = jnp.dot(xg_ref[...], w1x_ref[...], preferred_element_type=jnp.float32)
    z = z + jnp.dot(dp_ref[...], w1p_ref[...], preferred_element_type=jnp.float32)
    h1 = _lrelu(z + b1_ref[...])
    h2 = _lrelu(jnp.dot(h1, w2_ref[...], preferred_element_type=jnp.float32)
                + b2_ref[...])
    h_ref[...] = _lrelu(jnp.dot(h2, w3_ref[...], preferred_element_type=jnp.float32)
                        + b3_ref[...])


def _edge_mlp(xg, dp, W1, b1, W2, b2, W3, b3):
    w1x = W1[:D]
    w1p = jnp.pad(W1[D:], ((0, 1), (0, 0)))
    grid = (E // E_BLK,)
    return pl.pallas_call(
        _edge_mlp_body,
        grid=grid,
        in_specs=[
            pl.BlockSpec((E_BLK, D), lambda i: (i, 0)),
            pl.BlockSpec((E_BLK, 4), lambda i: (i, 0)),
            pl.BlockSpec((D, 64), lambda i: (0, 0)),
            pl.BlockSpec((4, 64), lambda i: (0, 0)),
            pl.BlockSpec((1, 64), lambda i: (0, 0)),
            pl.BlockSpec((64, 128), lambda i: (0, 0)),
            pl.BlockSpec((1, 128), lambda i: (0, 0)),
            pl.BlockSpec((128, 512), lambda i: (0, 0)),
            pl.BlockSpec((1, 512), lambda i: (0, 0)),
        ],
        out_specs=pl.BlockSpec((E_BLK, 512), lambda i: (i, 0)),
        out_shape=jax.ShapeDtypeStruct((E, 512), jnp.float32),
    )(xg, dp, w1x, w1p, b1.reshape(1, 64), W2, b2.reshape(1, 128),
      W3, b3.reshape(1, 512))


def _head_body(agg_ref, ps_ref, w4a_ref, w4p_ref, b4_ref, w5_ref, b5_ref,
               out_ref):
    agg = agg_ref[...]
    agg = jnp.where(jnp.isfinite(agg), agg, 0.0)
    z = jnp.dot(agg, w4a_ref[...], preferred_element_type=jnp.float32)
    z = z + jnp.dot(ps_ref[...], w4p_ref[...], preferred_element_type=jnp.float32)
    y1 = _lrelu(z + b4_ref[...])
    y = jnp.dot(y1, w5_ref[...], preferred_element_type=jnp.float32) + b5_ref[...]
    mean = y[:, :512]
    std = jnp.exp(0.5 * y[:, 512:])
    out_ref[...] = jnp.concatenate([mean, std], axis=-1)


def _head(agg, ps, W4, b4, W5, b5):
    w4a = W4[:512]
    w4p = jnp.pad(W4[512:], ((0, 1), (0, 0)))
    ps_pad = jnp.pad(ps, ((0, 0), (0, 1)))
    grid = (pl.cdiv(M, M_BLK),)
    return pl.pallas_call(
        _head_body,
        grid=grid,
        in_specs=[
            pl.BlockSpec((M_BLK, 512), lambda i: (i, 0)),
            pl.BlockSpec((M_BLK, 4), lambda i: (i, 0)),
            pl.BlockSpec((512, 512), lambda i: (0, 0)),
            pl.BlockSpec((4, 512), lambda i: (0, 0)),
            pl.BlockSpec((1, 512), lambda i: (0, 0)),
            pl.BlockSpec((512, 1024), lambda i: (0, 0)),
            pl.BlockSpec((1, 1024), lambda i: (0, 0)),
        ],
        out_specs=pl.BlockSpec((M_BLK, 1024), lambda i: (i, 0)),
        out_shape=jax.ShapeDtypeStruct((M, 1024), jnp.float32),
    )(agg, ps_pad, w4a, w4p, b4.reshape(1, 512), W5, b5.reshape(1, 1024))


@jax.jit
def kernel(x, pos, edge_index, idx, W1, b1, W2, b2, W3, b3, W4, b4, W5, b5):
    src = edge_index[0]
    dst = edge_index[1]
    ps = pos[idx]
    xg = x[src]
    dp = pos[src] - ps[dst]
    dp = jnp.pad(dp, ((0, 0), (0, 1)))
    h = _edge_mlp(xg, dp, W1, b1, W2, b2, W3, b3)
    agg = jax.ops.segment_max(h, dst, num_segments=M)
    return _head(agg, ps, W4, b4, W5, b5)


# speedup vs baseline: 1.2336x; 1.2336x over previous
import functools

import jax
import jax.numpy as jnp
from jax import lax
from jax.experimental import pallas as pl
from jax.experimental.pallas import tpu as pltpu
from jax.experimental.pallas import tpu_sc as plsc

N = 50000
E = 800000
M = 12500
D = 64

E_BLK = 2000
M_BLK = 2048


def _lrelu(v):
    return jnp.where(v > 0, v, 0.01 * v)


def _edge_mlp_body(xg_ref, dp_ref, w1x_ref, w1p_ref, b1_ref, w2_ref, b2_ref,
                   w3_ref, b3_ref, h_ref):
    z = jnp.dot(xg_ref[...], w1x_ref[...], preferred_element_type=jnp.float32)
    z = z + jnp.dot(dp_ref[...], w1p_ref[...], preferred_element_type=jnp.float32)
    h1 = _lrelu(z + b1_ref[...])
    h2 = _lrelu(jnp.dot(h1, w2_ref[...], preferred_element_type=jnp.float32)
                + b2_ref[...])
    h_ref[...] = _lrelu(jnp.dot(h2, w3_ref[...], preferred_element_type=jnp.float32)
                        + b3_ref[...]).astype(jnp.bfloat16)


def _edge_mlp(xg, dp, W1, b1, W2, b2, W3, b3):
    w1x = W1[:D]
    w1p = jnp.pad(W1[D:], ((0, 1), (0, 0)))
    grid = (E // E_BLK,)
    return pl.pallas_call(
        _edge_mlp_body,
        grid=grid,
        in_specs=[
            pl.BlockSpec((E_BLK, D), lambda i: (i, 0)),
            pl.BlockSpec((E_BLK, 4), lambda i: (i, 0)),
            pl.BlockSpec((D, 64), lambda i: (0, 0)),
            pl.BlockSpec((4, 64), lambda i: (0, 0)),
            pl.BlockSpec((1, 64), lambda i: (0, 0)),
            pl.BlockSpec((64, 128), lambda i: (0, 0)),
            pl.BlockSpec((1, 128), lambda i: (0, 0)),
            pl.BlockSpec((128, 512), lambda i: (0, 0)),
            pl.BlockSpec((1, 512), lambda i: (0, 0)),
        ],
        out_specs=pl.BlockSpec((E_BLK, 512), lambda i: (i, 0)),
        out_shape=jax.ShapeDtypeStruct((E, 512), jnp.bfloat16),
    )(xg, dp, w1x, w1p, b1.reshape(1, 64), W2, b2.reshape(1, 128),
      W3, b3.reshape(1, 512))


def _head_body(agg_ref, ps_ref, w4a_ref, w4p_ref, b4_ref, w5_ref, b5_ref,
               out_ref):
    agg = agg_ref[...]
    agg = jnp.where(jnp.isfinite(agg), agg, 0.0)
    z = jnp.dot(agg, w4a_ref[...], preferred_element_type=jnp.float32)
    z = z + jnp.dot(ps_ref[...], w4p_ref[...], preferred_element_type=jnp.float32)
    y1 = _lrelu(z + b4_ref[...])
    y = jnp.dot(y1, w5_ref[...], preferred_element_type=jnp.float32) + b5_ref[...]
    mean = y[:, :512]
    std = jnp.exp(0.5 * y[:, 512:])
    out_ref[...] = jnp.concatenate([mean, std], axis=-1)


def _head(agg, ps, W4, b4, W5, b5):
    w4a = W4[:512]
    w4p = jnp.pad(W4[512:], ((0, 1), (0, 0)))
    ps_pad = jnp.pad(ps, ((0, 0), (0, 1)))
    grid = (pl.cdiv(M, M_BLK),)
    return pl.pallas_call(
        _head_body,
        grid=grid,
        in_specs=[
            pl.BlockSpec((M_BLK, 512), lambda i: (i, 0)),
            pl.BlockSpec((M_BLK, 4), lambda i: (i, 0)),
            pl.BlockSpec((512, 512), lambda i: (0, 0)),
            pl.BlockSpec((4, 512), lambda i: (0, 0)),
            pl.BlockSpec((1, 512), lambda i: (0, 0)),
            pl.BlockSpec((512, 1024), lambda i: (0, 0)),
            pl.BlockSpec((1, 1024), lambda i: (0, 0)),
        ],
        out_specs=pl.BlockSpec((M_BLK, 1024), lambda i: (i, 0)),
        out_shape=jax.ShapeDtypeStruct((M, 1024), jnp.float32),
    )(agg, ps_pad, w4a, w4p, b4.reshape(1, 512), W5, b5.reshape(1, 1024))


@jax.jit
def kernel(x, pos, edge_index, idx, W1, b1, W2, b2, W3, b3, W4, b4, W5, b5):
    src = edge_index[0]
    dst = edge_index[1]
    ps = pos[idx]
    xg = x[src]
    dp = pos[src] - ps[dst]
    dp = jnp.pad(dp, ((0, 0), (0, 1)))
    h = _edge_mlp(xg, dp, W1, b1, W2, b2, W3, b3)
    agg = jax.ops.segment_max(h, dst, num_segments=M).astype(jnp.float32)
    return _head(agg, ps, W4, b4, W5, b5)
